# Initial kernel scaffold; baseline (speedup 1.0000x reference)
import jax
import jax.numpy as jnp
from jax.experimental import pallas as pl


def kernel(x, edge_index, batch, W1, b1, W2, b2, W3, b3, p1, pb1, p2, pb2, p3, pb3, l1W, l1b, l2W, l2b, l3W, l3b):
    raise NotImplementedError("write your pallas kernel here")


# speedup vs baseline: 1.1907x; 1.1907x over previous
import functools
import jax
import jax.numpy as jnp
from jax.experimental import pallas as pl

_N = 10000
_E = 320000
_H = 128
_B = 128
_C = 10
_RATIO = 0.5
_BLK = 2000


def _matmul_body(x_ref, w_ref, o_ref):
    o_ref[...] = jnp.dot(x_ref[...], w_ref[...],
                         preferred_element_type=jnp.float32)


def _matmul(x, w):
    n = x.shape[0]
    grid = n // _BLK if n % _BLK == 0 else 1
    blk = _BLK if n % _BLK == 0 else n
    return pl.pallas_call(
        _matmul_body,
        grid=(grid,),
        in_specs=[
            pl.BlockSpec((blk, x.shape[1]), lambda i: (i, 0)),
            pl.BlockSpec((w.shape[0], w.shape[1]), lambda i: (0, 0)),
        ],
        out_specs=pl.BlockSpec((blk, w.shape[1]), lambda i: (i, 0)),
        out_shape=jax.ShapeDtypeStruct((n, w.shape[1]), jnp.float32),
    )(x, w)


def _gcn_epilogue_body(agg_ref, hw_ref, scale_ref, mask_ref, b_ref, o_ref):
    o_ref[...] = (agg_ref[...] + hw_ref[...] * scale_ref[...]
                  + b_ref[...]) * mask_ref[...]


def _gcn_epilogue_relu_body(agg_ref, hw_ref, scale_ref, mask_ref, b_ref, o_ref):
    v = (agg_ref[...] + hw_ref[...] * scale_ref[...]
         + b_ref[...]) * mask_ref[...]
    o_ref[...] = jnp.maximum(v, 0.0)


def _gcn_epilogue(agg, hw, scale, mask, b, relu):
    body = _gcn_epilogue_relu_body if relu else _gcn_epilogue_body
    n = agg.shape[0]
    return pl.pallas_call(
        body,
        grid=(n // _BLK,),
        in_specs=[
            pl.BlockSpec((_BLK, _H), lambda i: (i, 0)),
            pl.BlockSpec((_BLK, _H), lambda i: (i, 0)),
            pl.BlockSpec((_BLK, 1), lambda i: (i, 0)),
            pl.BlockSpec((_BLK, 1), lambda i: (i, 0)),
            pl.BlockSpec((1, _H), lambda i: (0, 0)),
        ],
        out_specs=pl.BlockSpec((_BLK, _H), lambda i: (i, 0)),
        out_shape=jax.ShapeDtypeStruct((n, _H), jnp.float32),
    )(agg, hw, scale, mask, b)


def _readout_body(x_ref, mask_ref, oh_ref, mx_ref, sum_ref, cnt_ref):
    i = pl.program_id(0)

    @pl.when(i == 0)
    def _init():
        mx_ref[...] = jnp.full_like(mx_ref, -1e30)
        sum_ref[...] = jnp.zeros_like(sum_ref)
        cnt_ref[...] = jnp.zeros_like(cnt_ref)

    x = x_ref[...]
    m = mask_ref[...]
    oh = oh_ref[...] * m
    xm = x * m
    sum_ref[...] += jax.lax.dot_general(
        oh, xm, (((0,), (0,)), ((), ())),
        preferred_element_type=jnp.float32)
    cnt_ref[...] += jnp.sum(oh, axis=0, keepdims=True)

    def step(j, _):
        sub_x = x_ref[pl.ds(j * 50, 50), :]
        sub_oh = oh_ref[pl.ds(j * 50, 50), :] * mask_ref[pl.ds(j * 50, 50), :]
        big = jnp.where(sub_oh[:, :, None] > 0, sub_x[:, None, :], -1e30)
        mx_ref[...] = jnp.maximum(mx_ref[...], jnp.max(big, axis=0))
        return 0

    jax.lax.fori_loop(0, _BLK // 50, step, 0)


def _readout(x, mask, onehot):
    mx, sm, cnt = pl.pallas_call(
        _readout_body,
        grid=(_N // _BLK,),
        in_specs=[
            pl.BlockSpec((_BLK, _H), lambda i: (i, 0)),
            pl.BlockSpec((_BLK, 1), lambda i: (i, 0)),
            pl.BlockSpec((_BLK, _B), lambda i: (i, 0)),
        ],
        out_specs=[
            pl.BlockSpec((_B, _H), lambda i: (0, 0)),
            pl.BlockSpec((_B, _H), lambda i: (0, 0)),
            pl.BlockSpec((1, _B), lambda i: (0, 0)),
        ],
        out_shape=[
            jax.ShapeDtypeStruct((_B, _H), jnp.float32),
            jax.ShapeDtypeStruct((_B, _H), jnp.float32),
            jax.ShapeDtypeStruct((1, _B), jnp.float32),
        ],
    )(x, mask, onehot)
    mx = jnp.where(mx <= -1e29, 0.0, mx)
    mean = sm / jnp.maximum(cnt[0], 1.0)[:, None]
    return jnp.concatenate([mx, mean], axis=1)


def _mlp_body(z_ref, w1_ref, b1_ref, w2_ref, b2_ref, w3_ref, b3_ref, o_ref):
    z = jnp.maximum(jnp.dot(z_ref[...], w1_ref[...],
                            preferred_element_type=jnp.float32) + b1_ref[...], 0.0)
    z = jnp.maximum(jnp.dot(z, w2_ref[...],
                            preferred_element_type=jnp.float32) + b2_ref[...], 0.0)
    logits = jnp.dot(z, w3_ref[...],
                     preferred_element_type=jnp.float32) + b3_ref[...]
    col = jax.lax.broadcasted_iota(jnp.int32, logits.shape, 1)
    valid = col < _C
    neg = jnp.where(valid, logits, -1e30)
    m = jnp.max(neg, axis=1, keepdims=True)
    lse = m + jnp.log(jnp.sum(jnp.where(valid, jnp.exp(neg - m), 0.0),
                              axis=1, keepdims=True))
    o_ref[...] = logits - lse


def _mlp(z, l1W, l1b, l2W, l2b, l3W, l3b):
    w3p = jnp.pad(l3W, ((0, 0), (0, _H - _C)))
    b3p = jnp.pad(l3b, (0, _H - _C))
    out = pl.pallas_call(
        _mlp_body,
        in_specs=[
            pl.BlockSpec((_B, 2 * _H), lambda: (0, 0)),
            pl.BlockSpec((2 * _H, _H), lambda: (0, 0)),
            pl.BlockSpec((1, _H), lambda: (0, 0)),
            pl.BlockSpec((_H, _H // 2), lambda: (0, 0)),
            pl.BlockSpec((1, _H // 2), lambda: (0, 0)),
            pl.BlockSpec((_H // 2, _H), lambda: (0, 0)),
            pl.BlockSpec((1, _H), lambda: (0, 0)),
        ],
        out_specs=pl.BlockSpec((_B, _H), lambda: (0, 0)),
        out_shape=jax.ShapeDtypeStruct((_B, _H), jnp.float32),
    )(z, l1W, l1b.reshape(1, -1), l2W, l2b.reshape(1, -1), w3p,
      b3p.reshape(1, -1))
    return out[:, :_C]


def _edge_agg(hw, src, dst, ew, mask):
    deg = jax.ops.segment_sum(ew, dst, num_segments=_N) + mask
    deg_safe = jnp.where(deg > 0, deg, 1.0)
    inv = jax.lax.rsqrt(deg_safe)
    coef = ew * inv[src] * inv[dst]
    agg = jax.ops.segment_sum(hw[src] * coef[:, None], dst, num_segments=_N)
    return agg, deg_safe, coef


def _pool(x, score_raw, node_mask, batch, src, dst, seg_start):
    score = jnp.where(node_mask > 0, score_raw, -1e30)
    idx = jnp.argsort(-score)
    order = idx[jnp.argsort(batch[idx])]
    pos = jnp.arange(_N, dtype=jnp.int32)
    rank = jnp.zeros((_N,), jnp.int32).at[order].set(pos - seg_start[batch[order]])
    counts = jax.ops.segment_sum(node_mask, batch, num_segments=_B)
    keep = jnp.ceil(_RATIO * counts).astype(jnp.int32)
    new_mask = ((rank < keep[batch]) & (node_mask > 0)).astype(jnp.float32)
    x_new = x * jnp.tanh(score_raw)[:, None] * new_mask[:, None]
    edge_w = new_mask[src] * new_mask[dst]
    return x_new, new_mask, edge_w


def kernel(x, edge_index, batch, W1, b1, W2, b2, W3, b3, p1, pb1, p2, pb2,
           p3, pb3, l1W, l1b, l2W, l2b, l3W, l3b):
    src = edge_index[0]
    dst = edge_index[1]
    total = jax.ops.segment_sum(jnp.ones((_N,), jnp.float32), batch,
                                num_segments=_B)
    seg_start = (jnp.cumsum(total) - total).astype(jnp.int32)
    onehot = (batch[:, None] == jnp.arange(_B, dtype=jnp.int32)[None, :]
              ).astype(jnp.float32)
    mask = jnp.ones((_N,), jnp.float32)
    ew = jnp.ones((_E,), jnp.float32)

    h = x
    xs = []
    for (W, b, p, pb) in ((W1, b1, p1, pb1), (W2, b2, p2, pb2),
                          (W3, b3, p3, pb3)):
        hw = _matmul(h, W)
        agg, deg_safe, coef = _edge_agg(hw, src, dst, ew, mask)
        scale = (mask / deg_safe)[:, None]
        h = _gcn_epilogue(agg, hw, scale, mask[:, None],
                          b.reshape(1, -1), relu=True)
        hp = (h @ p)[:, 0]
        agg_s = jax.ops.segment_sum(hp[src] * coef, dst, num_segments=_N)
        s = (agg_s + hp * (mask / deg_safe) + pb[0]) * mask
        h, mask, ew = _pool(h, s, mask, batch, src, dst, seg_start)
        xs.append(_readout(h, mask[:, None], onehot))

    z = xs[0] + xs[1] + xs[2]
    return _mlp(z, l1W, l1b, l2W, l2b, l3W, l3b)


# speedup vs baseline: 1.1942x; 1.0030x over previous
import functools
import jax
import jax.numpy as jnp
from jax.experimental import pallas as pl

_N = 10000
_E = 320000
_H = 128
_B = 128
_C = 10
_RATIO = 0.5
_BLK = 2000


def _matmul_body(x_ref, w_ref, o_ref):
    o_ref[...] = jnp.dot(x_ref[...], w_ref[...],
                         preferred_element_type=jnp.float32)


def _matmul(x, w):
    n = x.shape[0]
    grid = n // _BLK if n % _BLK == 0 else 1
    blk = _BLK if n % _BLK == 0 else n
    return pl.pallas_call(
        _matmul_body,
        grid=(grid,),
        in_specs=[
            pl.BlockSpec((blk, x.shape[1]), lambda i: (i, 0)),
            pl.BlockSpec((w.shape[0], w.shape[1]), lambda i: (0, 0)),
        ],
        out_specs=pl.BlockSpec((blk, w.shape[1]), lambda i: (i, 0)),
        out_shape=jax.ShapeDtypeStruct((n, w.shape[1]), jnp.float32),
    )(x, w)


def _gcn_epilogue_body(agg_ref, hw_ref, scale_ref, mask_ref, b_ref, o_ref):
    o_ref[...] = (agg_ref[...] + hw_ref[...] * scale_ref[...]
                  + b_ref[...]) * mask_ref[...]


def _gcn_epilogue_relu_body(agg_ref, hw_ref, scale_ref, mask_ref, b_ref, o_ref):
    v = (agg_ref[...] + hw_ref[...] * scale_ref[...]
         + b_ref[...]) * mask_ref[...]
    o_ref[...] = jnp.maximum(v, 0.0)


def _gcn_epilogue(agg, hw, scale, mask, b, relu):
    body = _gcn_epilogue_relu_body if relu else _gcn_epilogue_body
    n = agg.shape[0]
    return pl.pallas_call(
        body,
        grid=(n // _BLK,),
        in_specs=[
            pl.BlockSpec((_BLK, _H), lambda i: (i, 0)),
            pl.BlockSpec((_BLK, _H), lambda i: (i, 0)),
            pl.BlockSpec((_BLK, 1), lambda i: (i, 0)),
            pl.BlockSpec((_BLK, 1), lambda i: (i, 0)),
            pl.BlockSpec((1, _H), lambda i: (0, 0)),
        ],
        out_specs=pl.BlockSpec((_BLK, _H), lambda i: (i, 0)),
        out_shape=jax.ShapeDtypeStruct((n, _H), jnp.float32),
    )(agg, hw, scale, mask, b)


def _readout_body(x_ref, mask_ref, oh_ref, mx_ref, sum_ref, cnt_ref):
    i = pl.program_id(0)

    @pl.when(i == 0)
    def _init():
        mx_ref[...] = jnp.full_like(mx_ref, -1e30)
        sum_ref[...] = jnp.zeros_like(sum_ref)
        cnt_ref[...] = jnp.zeros_like(cnt_ref)

    x = x_ref[...]
    m = mask_ref[...]
    oh = oh_ref[...] * m
    xm = x * m
    sum_ref[...] += jax.lax.dot_general(
        oh, xm, (((0,), (0,)), ((), ())),
        preferred_element_type=jnp.float32)
    cnt_ref[...] += jnp.sum(oh, axis=0, keepdims=True)

    def step(j, _):
        sub_x = x_ref[pl.ds(j * 50, 50), :]
        sub_oh = oh_ref[pl.ds(j * 50, 50), :] * mask_ref[pl.ds(j * 50, 50), :]
        big = jnp.where(sub_oh[:, :, None] > 0, sub_x[:, None, :], -1e30)
        mx_ref[...] = jnp.maximum(mx_ref[...], jnp.max(big, axis=0))
        return 0

    jax.lax.fori_loop(0, _BLK // 50, step, 0)


def _readout(x, mask, onehot):
    mx, sm, cnt = pl.pallas_call(
        _readout_body,
        grid=(_N // _BLK,),
        in_specs=[
            pl.BlockSpec((_BLK, _H), lambda i: (i, 0)),
            pl.BlockSpec((_BLK, 1), lambda i: (i, 0)),
            pl.BlockSpec((_BLK, _B), lambda i: (i, 0)),
        ],
        out_specs=[
            pl.BlockSpec((_B, _H), lambda i: (0, 0)),
            pl.BlockSpec((_B, _H), lambda i: (0, 0)),
            pl.BlockSpec((1, _B), lambda i: (0, 0)),
        ],
        out_shape=[
            jax.ShapeDtypeStruct((_B, _H), jnp.float32),
            jax.ShapeDtypeStruct((_B, _H), jnp.float32),
            jax.ShapeDtypeStruct((1, _B), jnp.float32),
        ],
    )(x, mask, onehot)
    mx = jnp.where(mx <= -1e29, 0.0, mx)
    mean = sm / jnp.maximum(cnt[0], 1.0)[:, None]
    return jnp.concatenate([mx, mean], axis=1)


def _mlp_body(z_ref, w1_ref, b1_ref, w2_ref, b2_ref, w3_ref, b3_ref, o_ref):
    z = jnp.maximum(jnp.dot(z_ref[...], w1_ref[...],
                            preferred_element_type=jnp.float32) + b1_ref[...], 0.0)
    z = jnp.maximum(jnp.dot(z, w2_ref[...],
                            preferred_element_type=jnp.float32) + b2_ref[...], 0.0)
    logits = jnp.dot(z, w3_ref[...],
                     preferred_element_type=jnp.float32) + b3_ref[...]
    col = jax.lax.broadcasted_iota(jnp.int32, logits.shape, 1)
    valid = col < _C
    neg = jnp.where(valid, logits, -1e30)
    m = jnp.max(neg, axis=1, keepdims=True)
    lse = m + jnp.log(jnp.sum(jnp.where(valid, jnp.exp(neg - m), 0.0),
                              axis=1, keepdims=True))
    o_ref[...] = logits - lse


def _mlp(z, l1W, l1b, l2W, l2b, l3W, l3b):
    w3p = jnp.pad(l3W, ((0, 0), (0, _H - _C)))
    b3p = jnp.pad(l3b, (0, _H - _C))
    out = pl.pallas_call(
        _mlp_body,
        in_specs=[
            pl.BlockSpec((_B, 2 * _H), lambda: (0, 0)),
            pl.BlockSpec((2 * _H, _H), lambda: (0, 0)),
            pl.BlockSpec((1, _H), lambda: (0, 0)),
            pl.BlockSpec((_H, _H // 2), lambda: (0, 0)),
            pl.BlockSpec((1, _H // 2), lambda: (0, 0)),
            pl.BlockSpec((_H // 2, _H), lambda: (0, 0)),
            pl.BlockSpec((1, _H), lambda: (0, 0)),
        ],
        out_specs=pl.BlockSpec((_B, _H), lambda: (0, 0)),
        out_shape=jax.ShapeDtypeStruct((_B, _H), jnp.float32),
    )(z, l1W, l1b.reshape(1, -1), l2W, l2b.reshape(1, -1), w3p,
      b3p.reshape(1, -1))
    return out[:, :_C]


_NP = 10240
_RIB = 2048
_RJB = 512


def _rank_body(sr_ref, br_ref, sc_ref, bc_ref, o_ref):
    i = pl.program_id(0)
    j = pl.program_id(1)

    @pl.when(j == 0)
    def _init():
        o_ref[...] = jnp.zeros_like(o_ref)

    brow = br_ref[...]
    bcol = bc_ref[...]
    overlap = jnp.logical_and(bcol[0, _RJB - 1] >= brow[0, 0],
                              bcol[0, 0] <= brow[_RIB - 1, 0])

    @pl.when(overlap)
    def _acc():
        srow = sr_ref[...]
        scol = sc_ref[...]
        same = brow == bcol
        gt = scol > srow
        ridx = i * _RIB + jax.lax.broadcasted_iota(jnp.int32, (_RIB, 1), 0)
        cidx = j * _RJB + jax.lax.broadcasted_iota(jnp.int32, (1, _RJB), 1)
        eqt = jnp.logical_and(scol == srow, cidx < ridx)
        contrib = jnp.logical_and(same, jnp.logical_or(gt, eqt))
        o_ref[...] += jnp.sum(contrib.astype(jnp.float32), axis=1,
                              keepdims=True)


def _rank(score_masked, batchf):
    sp = jnp.pad(score_masked, (0, _NP - _N), constant_values=-1e30)
    bp = jnp.pad(batchf, (0, _NP - _N), constant_values=-1.0)
    rank = pl.pallas_call(
        _rank_body,
        grid=(_NP // _RIB, _NP // _RJB),
        in_specs=[
            pl.BlockSpec((_RIB, 1), lambda i, j: (i, 0)),
            pl.BlockSpec((_RIB, 1), lambda i, j: (i, 0)),
            pl.BlockSpec((1, _RJB), lambda i, j: (0, j)),
            pl.BlockSpec((1, _RJB), lambda i, j: (0, j)),
        ],
        out_specs=pl.BlockSpec((_RIB, 1), lambda i, j: (i, 0)),
        out_shape=jax.ShapeDtypeStruct((_NP, 1), jnp.float32),
    )(sp.reshape(_NP, 1), bp.reshape(_NP, 1),
      sp.reshape(1, _NP), bp.reshape(1, _NP))
    return rank[:_N, 0]


def _edge_agg(hw, src, dst, ew, mask):
    deg = jax.ops.segment_sum(ew, dst, num_segments=_N) + mask
    deg_safe = jnp.where(deg > 0, deg, 1.0)
    inv = jax.lax.rsqrt(deg_safe)
    coef = ew * inv[src] * inv[dst]
    agg = jax.ops.segment_sum(hw[src] * coef[:, None], dst, num_segments=_N)
    return agg, deg_safe, coef


def _pool(x, score_raw, node_mask, batch, batchf, src, dst):
    score = jnp.where(node_mask > 0, score_raw, -1e30)
    rank = _rank(score, batchf)
    counts = jax.ops.segment_sum(node_mask, batch, num_segments=_B)
    keep = jnp.ceil(_RATIO * counts)
    new_mask = ((rank < keep[batch]) & (node_mask > 0)).astype(jnp.float32)
    x_new = x * jnp.tanh(score_raw)[:, None] * new_mask[:, None]
    edge_w = new_mask[src] * new_mask[dst]
    return x_new, new_mask, edge_w


def kernel(x, edge_index, batch, W1, b1, W2, b2, W3, b3, p1, pb1, p2, pb2,
           p3, pb3, l1W, l1b, l2W, l2b, l3W, l3b):
    src = edge_index[0]
    dst = edge_index[1]
    batchf = batch.astype(jnp.float32)
    onehot = (batch[:, None] == jnp.arange(_B, dtype=jnp.int32)[None, :]
              ).astype(jnp.float32)
    mask = jnp.ones((_N,), jnp.float32)
    ew = jnp.ones((_E,), jnp.float32)

    h = x
    xs = []
    for (W, b, p, pb) in ((W1, b1, p1, pb1), (W2, b2, p2, pb2),
                          (W3, b3, p3, pb3)):
        hw = _matmul(h, W)
        agg, deg_safe, coef = _edge_agg(hw, src, dst, ew, mask)
        scale = (mask / deg_safe)[:, None]
        h = _gcn_epilogue(agg, hw, scale, mask[:, None],
                          b.reshape(1, -1), relu=True)
        hp = (h @ p)[:, 0]
        agg_s = jax.ops.segment_sum(hp[src] * coef, dst, num_segments=_N)
        s = (agg_s + hp * (mask / deg_safe) + pb[0]) * mask
        h, mask, ew = _pool(h, s, mask, batch, batchf, src, dst)
        xs.append(_readout(h, mask[:, None], onehot))

    z = xs[0] + xs[1] + xs[2]
    return _mlp(z, l1W, l1b, l2W, l2b, l3W, l3b)
